# baseline (speedup 1.0000x reference)
import functools
import math

import jax
import jax.numpy as jnp
from jax.experimental import pallas as pl
from jax.experimental.pallas import tpu as pltpu

_CUTOFF = 5.0
_K = 32
_HID = 128
_NG = 50
_NL = 6
_PROJ = 256
_RB = 256
_RBL = 128
_T = 512
_TG = 256
_NGP = 64
_EB = _RBL * _K
_NEG_BIG = -jnp.inf


def _ssp(x):
    return jnp.maximum(x, 0.0) + jnp.log1p(jnp.exp(-jnp.abs(x))) - math.log(2.0)


def _edge_kernel(scal_ref, posq_ref, posT_ref, sqT_ref, batq_ref, batT_ref,
                 topi_ref, dist_ref, *, np_, t, k):
    b = pl.program_id(0)
    rb = posq_ref.shape[0]
    posq = posq_ref[...]
    sqq = jnp.sum(posq * posq, axis=1, keepdims=True)
    batq = batq_ref[...]
    rowid = b * rb + jax.lax.broadcasted_iota(jnp.int32, (rb, 1), 0)
    lo = scal_ref[0, b]
    hi = scal_ref[1, b]
    kio = jax.lax.broadcasted_iota(jnp.int32, (1, k), 1)
    colio = jax.lax.broadcasted_iota(jnp.int32, (1, k + t), 1)
    bigi = jnp.int32(np_ + t + k + 7)

    def tile_body(c, carry):
        tv, ti = carry
        base = pl.multiple_of(c * t, t)
        post = posT_ref[:, pl.ds(base, t)]
        sqc = sqT_ref[:, pl.ds(base, t)]
        batc = batT_ref[:, pl.ds(base, t)]
        dot = jax.lax.dot_general(
            posq, post, (((1,), (0,)), ((), ())),
            preferred_element_type=jnp.float32,
            precision=jax.lax.Precision.HIGHEST)
        d2 = sqq + sqc - 2.0 * dot
        colid = base + jax.lax.broadcasted_iota(jnp.int32, (rb, t), 1)
        valid = (batq == batc) & (rowid != colid) & (d2 <= _CUTOFF * _CUTOFF)
        neg = jnp.where(valid, -d2, _NEG_BIG)
        cv = jnp.concatenate([tv, neg], axis=1)
        ci = jnp.concatenate([ti, colid], axis=1)

        def sel_body(s, sc):
            cv, ntv, nti = sc
            m = jnp.max(cv, axis=1, keepdims=True)
            ism = cv == m
            j = jnp.min(jnp.where(ism, colio, bigi), axis=1, keepdims=True)
            selm = colio == j
            oi = jnp.max(jnp.where(selm, ci, 0), axis=1, keepdims=True)
            smask = kio == s
            ntv = jnp.where(smask, m, ntv)
            nti = jnp.where(smask, oi, nti)
            cv = jnp.where(selm, _NEG_BIG, cv)
            return cv, ntv, nti

        _, tv2, ti2 = jax.lax.fori_loop(
            0, k, sel_body,
            (cv, jnp.full((rb, k), _NEG_BIG, jnp.float32),
             jnp.zeros((rb, k), jnp.int32)))
        return tv2, ti2

    tv, ti = jax.lax.fori_loop(
        lo, hi, tile_body,
        (jnp.full((rb, k), _NEG_BIG, jnp.float32),
         jnp.zeros((rb, k), jnp.int32)))
    maskb = tv > -1e30
    dist = jnp.where(maskb, jnp.sqrt(jnp.maximum(-tv, 0.0)), -1.0)
    topi_ref[...] = ti
    dist_ref[...] = dist


def _block_ranges(batch, n, np_, rb, tile):
    nblk = np_ // rb
    first = jnp.minimum(jnp.arange(nblk, dtype=jnp.int32) * rb, n - 1)
    last = jnp.minimum(jnp.arange(nblk, dtype=jnp.int32) * rb + rb - 1, n - 1)
    lo = jnp.searchsorted(batch, batch[first], side="left").astype(jnp.int32)
    hi = jnp.searchsorted(batch, batch[last], side="right").astype(jnp.int32)
    lo_t = lo // tile
    hi_t = (hi + tile - 1) // tile
    pad_blk = (jnp.arange(nblk, dtype=jnp.int32) * rb) >= n
    lo_t = jnp.where(pad_blk, 0, lo_t)
    hi_t = jnp.where(pad_blk, 0, hi_t)
    return jnp.stack([lo_t, hi_t]).astype(jnp.int32), nblk


def _build_edges(pos, batch, interpret=False):
    n = pos.shape[0]
    np_ = ((n + _T - 1) // _T) * _T
    posp = jnp.zeros((np_, 8), jnp.float32).at[:n, :3].set(pos)
    posp = posp.at[n:, 0].set(1e4)
    batf = jnp.full((np_,), -1.0, jnp.float32).at[:n].set(batch.astype(jnp.float32))
    sq = jnp.sum(posp * posp, axis=1)
    scal, nblk = _block_ranges(batch, n, np_, _RB, _T)

    grid_spec = pltpu.PrefetchScalarGridSpec(
        num_scalar_prefetch=1,
        grid=(nblk,),
        in_specs=[
            pl.BlockSpec((_RB, 8), lambda b, s: (b, 0)),
            pl.BlockSpec((8, np_), lambda b, s: (0, 0)),
            pl.BlockSpec((1, np_), lambda b, s: (0, 0)),
            pl.BlockSpec((_RB, 1), lambda b, s: (b, 0)),
            pl.BlockSpec((1, np_), lambda b, s: (0, 0)),
        ],
        out_specs=[
            pl.BlockSpec((_RB, _K), lambda b, s: (b, 0)),
            pl.BlockSpec((_RB, _K), lambda b, s: (b, 0)),
        ],
    )
    topi, dist = pl.pallas_call(
        functools.partial(_edge_kernel, np_=np_, t=_T, k=_K),
        grid_spec=grid_spec,
        out_shape=[
            jax.ShapeDtypeStruct((np_, _K), jnp.int32),
            jax.ShapeDtypeStruct((np_, _K), jnp.float32),
        ],
        interpret=interpret,
    )(scal, posp, posp.T, sq[None, :], batf[:, None], batf[None, :])
    return topi, dist, np_


def _split16(x):
    hi = x.astype(jnp.bfloat16)
    lo = (x - hi.astype(jnp.float32)).astype(jnp.bfloat16)
    return hi, lo


def _init_kernel(zq_ref, emb_ref, l1_ref, h_ref, x1h_ref, x1l_ref):
    zq = zq_ref[...]
    nv = emb_ref.shape[0]
    vio = jax.lax.broadcasted_iota(jnp.int32, (1, nv), 1)
    oh = (zq == vio).astype(jnp.float32)
    h = jax.lax.dot_general(oh, emb_ref[...], (((1,), (0,)), ((), ())),
                            preferred_element_type=jnp.float32,
                            precision=jax.lax.Precision.HIGHEST)
    h_ref[...] = h
    x1 = jax.lax.dot_general(h, l1_ref[...], (((1,), (0,)), ((), ())),
                             preferred_element_type=jnp.float32,
                             precision=jax.lax.Precision.HIGHEST)
    x1h_ref[...], x1l_ref[...] = _split16(x1)


def _init_h(z, emb, l1w, np_, nblk, interpret=False):
    n = z.shape[0]
    nvp = ((emb.shape[0] + 7) // 8) * 8
    embp = jnp.zeros((nvp, _HID), jnp.float32).at[:emb.shape[0]].set(emb)
    zp = jnp.zeros((np_, 1), jnp.int32).at[:n, 0].set(z.astype(jnp.int32))
    return pl.pallas_call(
        _init_kernel,
        grid=(nblk,),
        in_specs=[
            pl.BlockSpec((_RB, 1), lambda b: (b, 0)),
            pl.BlockSpec((nvp, _HID), lambda b: (0, 0)),
            pl.BlockSpec((_HID, _HID), lambda b: (0, 0)),
        ],
        out_specs=[
            pl.BlockSpec((_RB, _HID), lambda b: (b, 0)),
            pl.BlockSpec((_RB, _HID), lambda b: (b, 0)),
            pl.BlockSpec((_RB, _HID), lambda b: (b, 0)),
        ],
        out_shape=[
            jax.ShapeDtypeStruct((np_, _HID), jnp.float32),
            jax.ShapeDtypeStruct((np_, _HID), jnp.bfloat16),
            jax.ShapeDtypeStruct((np_, _HID), jnp.bfloat16),
        ],
        interpret=interpret,
    )(zp, embp, l1w)


def _layers_kernel(scal_ref, d_ref, ti_ref, h0_ref, x10h_ref, x10l_ref,
                   offs_ref, w1_ref, b1_ref, w2_ref, b2_ref, l2w_ref, l2b_ref,
                   lw_ref, lb_ref, l1n_ref, hn_ref, x1h_s, x1l_s, h_s,
                   *, coeff, t, k):
    l = pl.program_id(0)
    b = pl.program_id(1)
    rb = h0_ref.shape[0]
    hp = jax.lax.Precision.DEFAULT
    rsl = jax.lax.rem(l, 2)
    wsl = 1 - rsl

    @pl.when((l == 0) & (b == 0))
    def _():
        x1h_s[0] = x10h_ref[...]
        x1l_s[0] = x10l_ref[...]

    ds = d_ref[...]
    maskb = ds >= 0.0
    d = jnp.where(maskb, ds, 1.0)
    cc = jnp.where(maskb, 0.5 * (jnp.cos(d * (math.pi / _CUTOFF)) + 1.0), 0.0)
    ea = jnp.exp(coeff * (d - offs_ref[...]) ** 2)
    tt = _ssp(jax.lax.dot_general(ea, w1_ref[0], (((1,), (0,)), ((), ())),
                                  preferred_element_type=jnp.float32,
                                  precision=hp) + b1_ref[0])
    w = jax.lax.dot_general(tt, w2_ref[0], (((1,), (0,)), ((), ())),
                            preferred_element_type=jnp.float32,
                            precision=hp) + b2_ref[0]
    w = w * cc
    ti = ti_ref[...]
    lo = scal_ref[0, b]
    hi = scal_ref[1, b]
    eb = ds.shape[0]

    def gbody(c, g):
        base = pl.multiple_of(c * t, t)
        x1th = x1h_s[rsl, pl.ds(base, t), :]
        x1tl = x1l_s[rsl, pl.ds(base, t), :]
        colio = base + jax.lax.broadcasted_iota(jnp.int32, (1, t), 1)
        oh = (ti == colio).astype(jnp.bfloat16)
        g = g + jax.lax.dot_general(oh, x1th, (((1,), (0,)), ((), ())),
                                    preferred_element_type=jnp.float32,
                                    precision=hp)
        g = g + jax.lax.dot_general(oh, x1tl, (((1,), (0,)), ((), ())),
                                    preferred_element_type=jnp.float32,
                                    precision=hp)
        return g

    g = jax.lax.fori_loop(lo, hi, gbody, jnp.zeros((eb, _HID), jnp.float32))
    msg = g * w
    agg = jnp.sum(msg.reshape(rb, k, _HID), axis=1)
    x3 = _ssp(jax.lax.dot_general(agg, l2w_ref[0], (((1,), (0,)), ((), ())),
                                  preferred_element_type=jnp.float32,
                                  precision=hp) + l2b_ref[0])
    hprev = jnp.where(l == 0, h0_ref[...], h_s[pl.ds(b * rb, rb), :])
    hn = hprev + jax.lax.dot_general(x3, lw_ref[0], (((1,), (0,)), ((), ())),
                                     preferred_element_type=jnp.float32,
                                     precision=hp) + lb_ref[0]
    hn_ref[...] = hn
    h_s[pl.ds(b * rb, rb), :] = hn
    x1n = jax.lax.dot_general(hn, l1n_ref[0], (((1,), (0,)), ((), ())),
                              preferred_element_type=jnp.float32,
                              precision=hp)
    x1nh, x1nl = _split16(x1n)
    x1h_s[wsl, pl.ds(b * rb, rb), :] = x1nh
    x1l_s[wsl, pl.ds(b * rb, rb), :] = x1nl


def _layers(scal, d_e, ti_e, h, x1h, x1l, offs, w1s, b1s, w2s, b2s, l2ws, l2bs,
            lws, lbs, l1s, coeff, np_, nblk, interpret=False):
    nl = _NL
    grid_spec = pltpu.PrefetchScalarGridSpec(
        num_scalar_prefetch=1,
        grid=(nl, nblk),
        in_specs=[
            pl.BlockSpec((_EB, 1), lambda l, b, s: (b, 0)),
            pl.BlockSpec((_EB, 1), lambda l, b, s: (b, 0)),
            pl.BlockSpec((_RBL, _HID), lambda l, b, s: (b, 0)),
            pl.BlockSpec((np_, _HID), lambda l, b, s: (0, 0)),
            pl.BlockSpec((np_, _HID), lambda l, b, s: (0, 0)),
            pl.BlockSpec((1, _NGP), lambda l, b, s: (0, 0)),
            pl.BlockSpec((1, _NGP, _HID), lambda l, b, s: (l, 0, 0)),
            pl.BlockSpec((1, 1, _HID), lambda l, b, s: (l, 0, 0)),
            pl.BlockSpec((1, _HID, _HID), lambda l, b, s: (l, 0, 0)),
            pl.BlockSpec((1, 1, _HID), lambda l, b, s: (l, 0, 0)),
            pl.BlockSpec((1, _HID, _HID), lambda l, b, s: (l, 0, 0)),
            pl.BlockSpec((1, 1, _HID), lambda l, b, s: (l, 0, 0)),
            pl.BlockSpec((1, _HID, _HID), lambda l, b, s: (l, 0, 0)),
            pl.BlockSpec((1, 1, _HID), lambda l, b, s: (l, 0, 0)),
            pl.BlockSpec((1, _HID, _HID), lambda l, b, s: ((l + 1) % nl, 0, 0)),
        ],
        out_specs=[
            pl.BlockSpec((_RBL, _HID), lambda l, b, s: (b, 0)),
        ],
        scratch_shapes=[
            pltpu.VMEM((2, np_, _HID), jnp.bfloat16),
            pltpu.VMEM((2, np_, _HID), jnp.bfloat16),
            pltpu.VMEM((np_, _HID), jnp.float32),
        ],
    )
    return pl.pallas_call(
        functools.partial(_layers_kernel, coeff=coeff, t=_TG, k=_K),
        grid_spec=grid_spec,
        out_shape=[
            jax.ShapeDtypeStruct((np_, _HID), jnp.float32),
        ],
        interpret=interpret,
    )(scal, d_e, ti_e, h, x1h, x1l, offs, w1s, b1s, w2s, b2s, l2ws, l2bs,
      lws, lbs, l1s)[0]


def _final_kernel(h_ref, pw_ref, pb_ref, g_ref, bb_ref, o_ref):
    y = jax.lax.dot_general(h_ref[...], pw_ref[...], (((1,), (0,)), ((), ())),
                            preferred_element_type=jnp.float32,
                            precision=jax.lax.Precision.HIGHEST) + pb_ref[...]
    mu = jnp.mean(y, axis=-1, keepdims=True)
    var = jnp.mean((y - mu) ** 2, axis=-1, keepdims=True)
    yn = (y - mu) / jnp.sqrt(var + 1e-5) * g_ref[...] + bb_ref[...]
    o_ref[...] = yn * jax.nn.sigmoid(yn)


def _final(h, pw, pb, g, bb, np_, nblk, interpret=False):
    return pl.pallas_call(
        _final_kernel,
        grid=(nblk,),
        in_specs=[
            pl.BlockSpec((_RB, _HID), lambda b: (b, 0)),
            pl.BlockSpec((_HID, _PROJ), lambda b: (0, 0)),
            pl.BlockSpec((1, _PROJ), lambda b: (0, 0)),
            pl.BlockSpec((1, _PROJ), lambda b: (0, 0)),
            pl.BlockSpec((1, _PROJ), lambda b: (0, 0)),
        ],
        out_specs=pl.BlockSpec((_RB, _PROJ), lambda b: (b, 0)),
        out_shape=jax.ShapeDtypeStruct((np_, _PROJ), jnp.float32),
        interpret=interpret,
    )(h, pw, pb[None, :], g[None, :], bb[None, :])


def _forward(z, pos, batch, emb, mlp_w1, mlp_b1, mlp_w2, mlp_b2, lin1_w,
             lin2_w, lin2_b, lin_w, lin_b, proj_w, proj_b, ln_g, ln_b,
             interpret=False):
    n = pos.shape[0]
    ng = mlp_w1.shape[1]
    offset = jnp.linspace(0.0, _CUTOFF, ng)
    import numpy as _np
    _step = float(_np.linspace(_np.float32(0.0), _np.float32(_CUTOFF), ng,
                               dtype=_np.float32)[1])
    coeff = -0.5 / _step ** 2
    offs = jnp.zeros((1, _NGP), jnp.float32).at[0, :ng].set(offset)

    topi, dist, np_ = _build_edges(pos, batch, interpret)
    nblk = np_ // _RB
    scal_l, nblk_l = _block_ranges(batch, n, np_, _RBL, _TG)
    d_e = dist.reshape(np_ * _K, 1)
    ti_e = topi.reshape(np_ * _K, 1)

    w1s = jnp.zeros((_NL, _NGP, _HID), jnp.float32).at[:, :ng].set(mlp_w1)
    h, x1h, x1l = _init_h(z, emb, lin1_w[0], np_, nblk, interpret)
    h = _layers(scal_l, d_e, ti_e, h, x1h, x1l, offs, w1s,
                mlp_b1[:, None, :], mlp_w2, mlp_b2[:, None, :],
                lin2_w, lin2_b[:, None, :], lin_w, lin_b[:, None, :],
                lin1_w, coeff, np_, nblk_l, interpret)
    out = _final(h, proj_w, proj_b, ln_g, ln_b, np_, nblk, interpret)
    return out[:n], batch


def kernel(z, pos, batch, emb, mlp_w1, mlp_b1, mlp_w2, mlp_b2, lin1_w,
           lin2_w, lin2_b, lin_w, lin_b, proj_w, proj_b, ln_g, ln_b):
    return _forward(z, pos, batch, emb, mlp_w1, mlp_b1, mlp_w2, mlp_b2,
                    lin1_w, lin2_w, lin2_b, lin_w, lin_b, proj_w, proj_b,
                    ln_g, ln_b)


# speedup vs baseline: 1.0491x; 1.0491x over previous
import functools
import math

import jax
import jax.numpy as jnp
from jax.experimental import pallas as pl
from jax.experimental.pallas import tpu as pltpu

_CUTOFF = 5.0
_K = 32
_HID = 128
_NG = 50
_NL = 6
_PROJ = 256
_RB = 256
_RBL = 128
_T = 512
_TG = 256
_NGP = 64
_EB = _RBL * _K
_NEG_BIG = -jnp.inf


def _ssp(x):
    return jnp.maximum(x, 0.0) + jnp.log1p(jnp.exp(-jnp.abs(x))) - math.log(2.0)


def _edge_kernel(scal_ref, posq_ref, posT_ref, sqT_ref, batq_ref, batT_ref,
                 topi_ref, dist_ref, *, np_, t, k):
    b = pl.program_id(0)
    rb = posq_ref.shape[0]
    posq = posq_ref[...]
    sqq = jnp.sum(posq * posq, axis=1, keepdims=True)
    batq = batq_ref[...]
    rowid = b * rb + jax.lax.broadcasted_iota(jnp.int32, (rb, 1), 0)
    lo = scal_ref[0, b]
    hi = scal_ref[1, b]
    kio = jax.lax.broadcasted_iota(jnp.int32, (1, k), 1)
    colio = jax.lax.broadcasted_iota(jnp.int32, (1, k + t), 1)
    bigi = jnp.int32(np_ + t + k + 7)

    def tile_body(c, carry):
        tv, ti = carry
        base = pl.multiple_of(c * t, t)
        post = posT_ref[:, pl.ds(base, t)]
        sqc = sqT_ref[:, pl.ds(base, t)]
        batc = batT_ref[:, pl.ds(base, t)]
        dot = jax.lax.dot_general(
            posq, post, (((1,), (0,)), ((), ())),
            preferred_element_type=jnp.float32,
            precision=jax.lax.Precision.HIGHEST)
        d2 = sqq + sqc - 2.0 * dot
        colid = base + jax.lax.broadcasted_iota(jnp.int32, (rb, t), 1)
        valid = (batq == batc) & (rowid != colid) & (d2 <= _CUTOFF * _CUTOFF)
        neg = jnp.where(valid, -d2, _NEG_BIG)
        cv = jnp.concatenate([tv, neg], axis=1)
        ci = jnp.concatenate([ti, colid], axis=1)

        def sel_body(s, sc):
            cv, ntv, nti = sc
            m = jnp.max(cv, axis=1, keepdims=True)
            ism = cv == m
            j = jnp.min(jnp.where(ism, colio, bigi), axis=1, keepdims=True)
            selm = colio == j
            oi = jnp.max(jnp.where(selm, ci, 0), axis=1, keepdims=True)
            smask = kio == s
            ntv = jnp.where(smask, m, ntv)
            nti = jnp.where(smask, oi, nti)
            cv = jnp.where(selm, _NEG_BIG, cv)
            return cv, ntv, nti

        _, tv2, ti2 = jax.lax.fori_loop(
            0, k, sel_body,
            (cv, jnp.full((rb, k), _NEG_BIG, jnp.float32),
             jnp.zeros((rb, k), jnp.int32)))
        return tv2, ti2

    tv, ti = jax.lax.fori_loop(
        lo, hi, tile_body,
        (jnp.full((rb, k), _NEG_BIG, jnp.float32),
         jnp.zeros((rb, k), jnp.int32)))
    maskb = tv > -1e30
    dist = jnp.where(maskb, jnp.sqrt(jnp.maximum(-tv, 0.0)), -1.0)
    topi_ref[...] = ti
    dist_ref[...] = dist


def _block_ranges(batch, n, np_, rb, tile):
    nblk = np_ // rb
    first = jnp.minimum(jnp.arange(nblk, dtype=jnp.int32) * rb, n - 1)
    last = jnp.minimum(jnp.arange(nblk, dtype=jnp.int32) * rb + rb - 1, n - 1)
    lo = jnp.searchsorted(batch, batch[first], side="left").astype(jnp.int32)
    hi = jnp.searchsorted(batch, batch[last], side="right").astype(jnp.int32)
    lo_t = lo // tile
    hi_t = (hi + tile - 1) // tile
    pad_blk = (jnp.arange(nblk, dtype=jnp.int32) * rb) >= n
    lo_t = jnp.where(pad_blk, 0, lo_t)
    hi_t = jnp.where(pad_blk, 0, hi_t)
    return jnp.stack([lo_t, hi_t]).astype(jnp.int32), nblk


def _build_edges(pos, batch, interpret=False):
    n = pos.shape[0]
    np_ = ((n + _T - 1) // _T) * _T
    posp = jnp.zeros((np_, 8), jnp.float32).at[:n, :3].set(pos)
    posp = posp.at[n:, 0].set(1e4)
    batf = jnp.full((np_,), -1.0, jnp.float32).at[:n].set(batch.astype(jnp.float32))
    sq = jnp.sum(posp * posp, axis=1)
    scal, nblk = _block_ranges(batch, n, np_, _RB, _T)

    grid_spec = pltpu.PrefetchScalarGridSpec(
        num_scalar_prefetch=1,
        grid=(nblk,),
        in_specs=[
            pl.BlockSpec((_RB, 8), lambda b, s: (b, 0)),
            pl.BlockSpec((8, np_), lambda b, s: (0, 0)),
            pl.BlockSpec((1, np_), lambda b, s: (0, 0)),
            pl.BlockSpec((_RB, 1), lambda b, s: (b, 0)),
            pl.BlockSpec((1, np_), lambda b, s: (0, 0)),
        ],
        out_specs=[
            pl.BlockSpec((_RB, _K), lambda b, s: (b, 0)),
            pl.BlockSpec((_RB, _K), lambda b, s: (b, 0)),
        ],
    )
    topi, dist = pl.pallas_call(
        functools.partial(_edge_kernel, np_=np_, t=_T, k=_K),
        grid_spec=grid_spec,
        out_shape=[
            jax.ShapeDtypeStruct((np_, _K), jnp.int32),
            jax.ShapeDtypeStruct((np_, _K), jnp.float32),
        ],
        interpret=interpret,
    )(scal, posp, posp.T, sq[None, :], batf[:, None], batf[None, :])
    return topi, dist, np_


def _split16(x):
    hi = x.astype(jnp.bfloat16)
    lo = (x - hi.astype(jnp.float32)).astype(jnp.bfloat16)
    return hi, lo


def _init_kernel(zq_ref, emb_ref, l1_ref, h_ref, x1h_ref):
    zq = zq_ref[...]
    nv = emb_ref.shape[0]
    vio = jax.lax.broadcasted_iota(jnp.int32, (1, nv), 1)
    oh = (zq == vio).astype(jnp.float32)
    h = jax.lax.dot_general(oh, emb_ref[...], (((1,), (0,)), ((), ())),
                            preferred_element_type=jnp.float32,
                            precision=jax.lax.Precision.HIGHEST)
    h_ref[...] = h
    x1 = jax.lax.dot_general(h, l1_ref[...], (((1,), (0,)), ((), ())),
                             preferred_element_type=jnp.float32,
                             precision=jax.lax.Precision.HIGHEST)
    x1h_ref[...] = x1.astype(jnp.bfloat16)


def _init_h(z, emb, l1w, np_, nblk, interpret=False):
    n = z.shape[0]
    nvp = ((emb.shape[0] + 7) // 8) * 8
    embp = jnp.zeros((nvp, _HID), jnp.float32).at[:emb.shape[0]].set(emb)
    zp = jnp.zeros((np_, 1), jnp.int32).at[:n, 0].set(z.astype(jnp.int32))
    return pl.pallas_call(
        _init_kernel,
        grid=(nblk,),
        in_specs=[
            pl.BlockSpec((_RB, 1), lambda b: (b, 0)),
            pl.BlockSpec((nvp, _HID), lambda b: (0, 0)),
            pl.BlockSpec((_HID, _HID), lambda b: (0, 0)),
        ],
        out_specs=[
            pl.BlockSpec((_RB, _HID), lambda b: (b, 0)),
            pl.BlockSpec((_RB, _HID), lambda b: (b, 0)),
        ],
        out_shape=[
            jax.ShapeDtypeStruct((np_, _HID), jnp.float32),
            jax.ShapeDtypeStruct((np_, _HID), jnp.bfloat16),
        ],
        interpret=interpret,
    )(zp, embp, l1w)


def _layers_kernel(scal_ref, d_ref, ti_ref, h0_ref, x10h_ref,
                   offs_ref, w1_ref, b1_ref, w2_ref, b2_ref, l2w_ref, l2b_ref,
                   lw_ref, lb_ref, l1n_ref, hn_ref, x1h_s, h_s,
                   *, coeff, t, k):
    l = pl.program_id(0)
    b = pl.program_id(1)
    rb = h0_ref.shape[0]
    hp = jax.lax.Precision.DEFAULT
    rsl = jax.lax.rem(l, 2)
    wsl = 1 - rsl

    @pl.when((l == 0) & (b == 0))
    def _():
        x1h_s[0] = x10h_ref[...]

    ds = d_ref[...]
    maskb = ds >= 0.0
    d = jnp.where(maskb, ds, 1.0)
    cc = jnp.where(maskb, 0.5 * (jnp.cos(d * (math.pi / _CUTOFF)) + 1.0), 0.0)
    ea = jnp.exp(coeff * (d - offs_ref[...]) ** 2).astype(jnp.bfloat16)
    tt = _ssp(jax.lax.dot_general(ea, w1_ref[0], (((1,), (0,)), ((), ())),
                                  preferred_element_type=jnp.float32,
                                  precision=hp) + b1_ref[0])
    w = jax.lax.dot_general(tt.astype(jnp.bfloat16), w2_ref[0],
                            (((1,), (0,)), ((), ())),
                            preferred_element_type=jnp.float32,
                            precision=hp) + b2_ref[0]
    w = w * cc
    ti = ti_ref[...]
    lo = scal_ref[0, b]
    hi = scal_ref[1, b]
    eb = ds.shape[0]

    def gbody(c, g):
        base = pl.multiple_of(c * t, t)
        x1th = x1h_s[rsl, pl.ds(base, t), :]
        colio = base + jax.lax.broadcasted_iota(jnp.int32, (1, t), 1)
        oh = (ti == colio).astype(jnp.bfloat16)
        g = g + jax.lax.dot_general(oh, x1th, (((1,), (0,)), ((), ())),
                                    preferred_element_type=jnp.float32,
                                    precision=hp)
        return g

    g = jax.lax.fori_loop(lo, hi, gbody, jnp.zeros((eb, _HID), jnp.float32))
    msg = g * w
    agg = jnp.sum(msg.reshape(rb, k, _HID), axis=1)
    x3 = _ssp(jax.lax.dot_general(agg, l2w_ref[0], (((1,), (0,)), ((), ())),
                                  preferred_element_type=jnp.float32,
                                  precision=hp) + l2b_ref[0])
    hprev = jnp.where(l == 0, h0_ref[...], h_s[pl.ds(b * rb, rb), :])
    hn = hprev + jax.lax.dot_general(x3, lw_ref[0], (((1,), (0,)), ((), ())),
                                     preferred_element_type=jnp.float32,
                                     precision=hp) + lb_ref[0]
    hn_ref[...] = hn
    h_s[pl.ds(b * rb, rb), :] = hn
    x1n = jax.lax.dot_general(hn, l1n_ref[0], (((1,), (0,)), ((), ())),
                              preferred_element_type=jnp.float32,
                              precision=hp)
    x1h_s[wsl, pl.ds(b * rb, rb), :] = x1n.astype(jnp.bfloat16)


def _layers(scal, d_e, ti_e, h, x1h, offs, w1s, b1s, w2s, b2s, l2ws, l2bs,
            lws, lbs, l1s, coeff, np_, nblk, interpret=False):
    nl = _NL
    grid_spec = pltpu.PrefetchScalarGridSpec(
        num_scalar_prefetch=1,
        grid=(nl, nblk),
        in_specs=[
            pl.BlockSpec((_EB, 1), lambda l, b, s: (b, 0)),
            pl.BlockSpec((_EB, 1), lambda l, b, s: (b, 0)),
            pl.BlockSpec((_RBL, _HID), lambda l, b, s: (b, 0)),
            pl.BlockSpec((np_, _HID), lambda l, b, s: (0, 0)),
            pl.BlockSpec((1, _NGP), lambda l, b, s: (0, 0)),
            pl.BlockSpec((1, _NGP, _HID), lambda l, b, s: (l, 0, 0)),
            pl.BlockSpec((1, 1, _HID), lambda l, b, s: (l, 0, 0)),
            pl.BlockSpec((1, _HID, _HID), lambda l, b, s: (l, 0, 0)),
            pl.BlockSpec((1, 1, _HID), lambda l, b, s: (l, 0, 0)),
            pl.BlockSpec((1, _HID, _HID), lambda l, b, s: (l, 0, 0)),
            pl.BlockSpec((1, 1, _HID), lambda l, b, s: (l, 0, 0)),
            pl.BlockSpec((1, _HID, _HID), lambda l, b, s: (l, 0, 0)),
            pl.BlockSpec((1, 1, _HID), lambda l, b, s: (l, 0, 0)),
            pl.BlockSpec((1, _HID, _HID), lambda l, b, s: ((l + 1) % nl, 0, 0)),
        ],
        out_specs=[
            pl.BlockSpec((_RBL, _HID), lambda l, b, s: (b, 0)),
        ],
        scratch_shapes=[
            pltpu.VMEM((2, np_, _HID), jnp.bfloat16),
            pltpu.VMEM((np_, _HID), jnp.float32),
        ],
    )
    return pl.pallas_call(
        functools.partial(_layers_kernel, coeff=coeff, t=_TG, k=_K),
        grid_spec=grid_spec,
        out_shape=[
            jax.ShapeDtypeStruct((np_, _HID), jnp.float32),
        ],
        interpret=interpret,
    )(scal, d_e, ti_e, h, x1h, offs, w1s, b1s, w2s, b2s, l2ws, l2bs,
      lws, lbs, l1s)[0]


def _final_kernel(h_ref, pw_ref, pb_ref, g_ref, bb_ref, o_ref):
    y = jax.lax.dot_general(h_ref[...], pw_ref[...], (((1,), (0,)), ((), ())),
                            preferred_element_type=jnp.float32,
                            precision=jax.lax.Precision.HIGHEST) + pb_ref[...]
    mu = jnp.mean(y, axis=-1, keepdims=True)
    var = jnp.mean((y - mu) ** 2, axis=-1, keepdims=True)
    yn = (y - mu) / jnp.sqrt(var + 1e-5) * g_ref[...] + bb_ref[...]
    o_ref[...] = yn * jax.nn.sigmoid(yn)


def _final(h, pw, pb, g, bb, np_, nblk, interpret=False):
    return pl.pallas_call(
        _final_kernel,
        grid=(nblk,),
        in_specs=[
            pl.BlockSpec((_RB, _HID), lambda b: (b, 0)),
            pl.BlockSpec((_HID, _PROJ), lambda b: (0, 0)),
            pl.BlockSpec((1, _PROJ), lambda b: (0, 0)),
            pl.BlockSpec((1, _PROJ), lambda b: (0, 0)),
            pl.BlockSpec((1, _PROJ), lambda b: (0, 0)),
        ],
        out_specs=pl.BlockSpec((_RB, _PROJ), lambda b: (b, 0)),
        out_shape=jax.ShapeDtypeStruct((np_, _PROJ), jnp.float32),
        interpret=interpret,
    )(h, pw, pb[None, :], g[None, :], bb[None, :])


def _forward(z, pos, batch, emb, mlp_w1, mlp_b1, mlp_w2, mlp_b2, lin1_w,
             lin2_w, lin2_b, lin_w, lin_b, proj_w, proj_b, ln_g, ln_b,
             interpret=False):
    n = pos.shape[0]
    ng = mlp_w1.shape[1]
    offset = jnp.linspace(0.0, _CUTOFF, ng)
    import numpy as _np
    _step = float(_np.linspace(_np.float32(0.0), _np.float32(_CUTOFF), ng,
                               dtype=_np.float32)[1])
    coeff = -0.5 / _step ** 2
    offs = jnp.zeros((1, _NGP), jnp.float32).at[0, :ng].set(offset)

    topi, dist, np_ = _build_edges(pos, batch, interpret)
    nblk = np_ // _RB
    scal_l, nblk_l = _block_ranges(batch, n, np_, _RBL, _TG)
    d_e = dist.reshape(np_ * _K, 1)
    ti_e = topi.reshape(np_ * _K, 1)

    w1s = jnp.zeros((_NL, _NGP, _HID), jnp.float32).at[:, :ng].set(mlp_w1)
    w1s = w1s.astype(jnp.bfloat16)
    w2s = mlp_w2.astype(jnp.bfloat16)
    h, x1h = _init_h(z, emb, lin1_w[0], np_, nblk, interpret)
    h = _layers(scal_l, d_e, ti_e, h, x1h, offs, w1s,
                mlp_b1[:, None, :], w2s, mlp_b2[:, None, :],
                lin2_w, lin2_b[:, None, :], lin_w, lin_b[:, None, :],
                lin1_w, coeff, np_, nblk_l, interpret)
    out = _final(h, proj_w, proj_b, ln_g, ln_b, np_, nblk, interpret)
    return out[:n], batch


def kernel(z, pos, batch, emb, mlp_w1, mlp_b1, mlp_w2, mlp_b2, lin1_w,
           lin2_w, lin2_b, lin_w, lin_b, proj_w, proj_b, ln_g, ln_b):
    return _forward(z, pos, batch, emb, mlp_w1, mlp_b1, mlp_w2, mlp_b2,
                    lin1_w, lin2_w, lin2_b, lin_w, lin_b, proj_w, proj_b,
                    ln_g, ln_b)


# speedup vs baseline: 1.0831x; 1.0324x over previous
import functools
import math

import jax
import jax.numpy as jnp
from jax.experimental import pallas as pl
from jax.experimental.pallas import tpu as pltpu

_CUTOFF = 5.0
_K = 32
_HID = 128
_NG = 50
_NL = 6
_PROJ = 256
_RB = 256
_RBL = 128
_T = 512
_TG = 256
_NGP = 64
_EB = _RBL * _K
_NEG_BIG = -jnp.inf


def _ssp(x):
    return jnp.maximum(x, 0.0) + jnp.log1p(jnp.exp(-jnp.abs(x))) - math.log(2.0)


def _edge_kernel(scal_ref, posq_ref, posT_ref, sqT_ref, batq_ref, batT_ref,
                 offs_ref, topi_ref, dist_ref, ea_ref, *, np_, t, k, coeff):
    b = pl.program_id(0)
    rb = posq_ref.shape[0]
    posq = posq_ref[...]
    sqq = jnp.sum(posq * posq, axis=1, keepdims=True)
    batq = batq_ref[...]
    rowid = b * rb + jax.lax.broadcasted_iota(jnp.int32, (rb, 1), 0)
    lo = scal_ref[0, b]
    hi = scal_ref[1, b]
    kio = jax.lax.broadcasted_iota(jnp.int32, (1, k), 1)
    colio = jax.lax.broadcasted_iota(jnp.int32, (1, k + t), 1)
    bigi = jnp.int32(np_ + t + k + 7)

    def tile_body(c, carry):
        tv, ti = carry
        base = pl.multiple_of(c * t, t)
        post = posT_ref[:, pl.ds(base, t)]
        sqc = sqT_ref[:, pl.ds(base, t)]
        batc = batT_ref[:, pl.ds(base, t)]
        dot = jax.lax.dot_general(
            posq, post, (((1,), (0,)), ((), ())),
            preferred_element_type=jnp.float32,
            precision=jax.lax.Precision.HIGHEST)
        d2 = sqq + sqc - 2.0 * dot
        colid = base + jax.lax.broadcasted_iota(jnp.int32, (rb, t), 1)
        valid = (batq == batc) & (rowid != colid) & (d2 <= _CUTOFF * _CUTOFF)
        neg = jnp.where(valid, -d2, _NEG_BIG)
        cv = jnp.concatenate([tv, neg], axis=1)
        ci = jnp.concatenate([ti, colid], axis=1)

        def sel_body(s, sc):
            cv, ntv, nti = sc
            m = jnp.max(cv, axis=1, keepdims=True)
            ism = cv == m
            j = jnp.min(jnp.where(ism, colio, bigi), axis=1, keepdims=True)
            selm = colio == j
            oi = jnp.max(jnp.where(selm, ci, 0), axis=1, keepdims=True)
            smask = kio == s
            ntv = jnp.where(smask, m, ntv)
            nti = jnp.where(smask, oi, nti)
            cv = jnp.where(selm, _NEG_BIG, cv)
            return cv, ntv, nti

        _, tv2, ti2 = jax.lax.fori_loop(
            0, k, sel_body,
            (cv, jnp.full((rb, k), _NEG_BIG, jnp.float32),
             jnp.zeros((rb, k), jnp.int32)))
        return tv2, ti2

    tv, ti = jax.lax.fori_loop(
        lo, hi, tile_body,
        (jnp.full((rb, k), _NEG_BIG, jnp.float32),
         jnp.zeros((rb, k), jnp.int32)))
    maskb = tv > -1e30
    dist = jnp.where(maskb, jnp.sqrt(jnp.maximum(-tv, 0.0)), -1.0)
    topi_ref[...] = ti
    dist_ref[...] = dist
    off3 = jnp.reshape(offs_ref[...], (1, 1, offs_ref.shape[1]))
    ea3 = jnp.exp(coeff * (dist[:, :, None] - off3) ** 2)
    ea_ref[...] = ea3.reshape(rb * k, -1).astype(jnp.bfloat16)


def _block_ranges(batch, n, np_, rb, tile):
    nblk = np_ // rb
    first = jnp.minimum(jnp.arange(nblk, dtype=jnp.int32) * rb, n - 1)
    last = jnp.minimum(jnp.arange(nblk, dtype=jnp.int32) * rb + rb - 1, n - 1)
    lo = jnp.searchsorted(batch, batch[first], side="left").astype(jnp.int32)
    hi = jnp.searchsorted(batch, batch[last], side="right").astype(jnp.int32)
    lo_t = lo // tile
    hi_t = (hi + tile - 1) // tile
    pad_blk = (jnp.arange(nblk, dtype=jnp.int32) * rb) >= n
    lo_t = jnp.where(pad_blk, 0, lo_t)
    hi_t = jnp.where(pad_blk, 0, hi_t)
    return jnp.stack([lo_t, hi_t]).astype(jnp.int32), nblk


def _build_edges(pos, batch, offs, coeff, interpret=False):
    n = pos.shape[0]
    np_ = ((n + _T - 1) // _T) * _T
    posp = jnp.zeros((np_, 8), jnp.float32).at[:n, :3].set(pos)
    posp = posp.at[n:, 0].set(1e4)
    batf = jnp.full((np_,), -1.0, jnp.float32).at[:n].set(batch.astype(jnp.float32))
    sq = jnp.sum(posp * posp, axis=1)
    scal, nblk = _block_ranges(batch, n, np_, _RB, _T)

    grid_spec = pltpu.PrefetchScalarGridSpec(
        num_scalar_prefetch=1,
        grid=(nblk,),
        in_specs=[
            pl.BlockSpec((_RB, 8), lambda b, s: (b, 0)),
            pl.BlockSpec((8, np_), lambda b, s: (0, 0)),
            pl.BlockSpec((1, np_), lambda b, s: (0, 0)),
            pl.BlockSpec((_RB, 1), lambda b, s: (b, 0)),
            pl.BlockSpec((1, np_), lambda b, s: (0, 0)),
            pl.BlockSpec((1, _NGP), lambda b, s: (0, 0)),
        ],
        out_specs=[
            pl.BlockSpec((_RB, _K), lambda b, s: (b, 0)),
            pl.BlockSpec((_RB, _K), lambda b, s: (b, 0)),
            pl.BlockSpec((_RB * _K, _NGP), lambda b, s: (b, 0)),
        ],
    )
    topi, dist, ea = pl.pallas_call(
        functools.partial(_edge_kernel, np_=np_, t=_T, k=_K, coeff=coeff),
        grid_spec=grid_spec,
        out_shape=[
            jax.ShapeDtypeStruct((np_, _K), jnp.int32),
            jax.ShapeDtypeStruct((np_, _K), jnp.float32),
            jax.ShapeDtypeStruct((np_ * _K, _NGP), jnp.bfloat16),
        ],
        interpret=interpret,
    )(scal, posp, posp.T, sq[None, :], batf[:, None], batf[None, :], offs)
    return topi, dist, ea, np_


def _split16(x):
    hi = x.astype(jnp.bfloat16)
    lo = (x - hi.astype(jnp.float32)).astype(jnp.bfloat16)
    return hi, lo


def _init_kernel(zq_ref, emb_ref, l1_ref, h_ref, x1h_ref):
    zq = zq_ref[...]
    nv = emb_ref.shape[0]
    vio = jax.lax.broadcasted_iota(jnp.int32, (1, nv), 1)
    oh = (zq == vio).astype(jnp.float32)
    h = jax.lax.dot_general(oh, emb_ref[...], (((1,), (0,)), ((), ())),
                            preferred_element_type=jnp.float32,
                            precision=jax.lax.Precision.HIGHEST)
    h_ref[...] = h
    x1 = jax.lax.dot_general(h, l1_ref[...], (((1,), (0,)), ((), ())),
                             preferred_element_type=jnp.float32,
                             precision=jax.lax.Precision.HIGHEST)
    x1h_ref[...] = x1.astype(jnp.bfloat16)


def _init_h(z, emb, l1w, np_, nblk, interpret=False):
    n = z.shape[0]
    nvp = ((emb.shape[0] + 7) // 8) * 8
    embp = jnp.zeros((nvp, _HID), jnp.float32).at[:emb.shape[0]].set(emb)
    zp = jnp.zeros((np_, 1), jnp.int32).at[:n, 0].set(z.astype(jnp.int32))
    return pl.pallas_call(
        _init_kernel,
        grid=(nblk,),
        in_specs=[
            pl.BlockSpec((_RB, 1), lambda b: (b, 0)),
            pl.BlockSpec((nvp, _HID), lambda b: (0, 0)),
            pl.BlockSpec((_HID, _HID), lambda b: (0, 0)),
        ],
        out_specs=[
            pl.BlockSpec((_RB, _HID), lambda b: (b, 0)),
            pl.BlockSpec((_RB, _HID), lambda b: (b, 0)),
        ],
        out_shape=[
            jax.ShapeDtypeStruct((np_, _HID), jnp.float32),
            jax.ShapeDtypeStruct((np_, _HID), jnp.bfloat16),
        ],
        interpret=interpret,
    )(zp, embp, l1w)


def _layers_kernel(scal_ref, d_ref, ti_ref, ea_ref, h0_ref, x10h_ref,
                   w1_ref, b1_ref, w2_ref, b2_ref, l2w_ref, l2b_ref,
                   lw_ref, lb_ref, l1n_ref, hn_ref, x1h_s, h_s,
                   *, t, k):
    l = pl.program_id(0)
    b = pl.program_id(1)
    rb = h0_ref.shape[0]
    hp = jax.lax.Precision.DEFAULT
    rsl = jax.lax.rem(l, 2)
    wsl = 1 - rsl

    @pl.when((l == 0) & (b == 0))
    def _():
        x1h_s[0] = x10h_ref[...]

    ds = d_ref[...]
    maskb = ds >= 0.0
    d = jnp.where(maskb, ds, 1.0)
    cc = jnp.where(maskb, 0.5 * (jnp.cos(d * (math.pi / _CUTOFF)) + 1.0), 0.0)
    ea = ea_ref[...]
    tt = _ssp(jax.lax.dot_general(ea, w1_ref[0], (((1,), (0,)), ((), ())),
                                  preferred_element_type=jnp.float32,
                                  precision=hp) + b1_ref[0])
    w = jax.lax.dot_general(tt.astype(jnp.bfloat16), w2_ref[0],
                            (((1,), (0,)), ((), ())),
                            preferred_element_type=jnp.float32,
                            precision=hp) + b2_ref[0]
    w = w * cc
    ti = ti_ref[...]
    lo = scal_ref[0, b]
    hi = scal_ref[1, b]
    eb = ds.shape[0]

    def gbody(c, g):
        base = pl.multiple_of(c * t, t)
        x1th = x1h_s[rsl, pl.ds(base, t), :]
        colio = base + jax.lax.broadcasted_iota(jnp.int32, (1, t), 1)
        oh = (ti == colio).astype(jnp.bfloat16)
        g = g + jax.lax.dot_general(oh, x1th, (((1,), (0,)), ((), ())),
                                    preferred_element_type=jnp.float32,
                                    precision=hp)
        return g

    g = jax.lax.fori_loop(lo, hi, gbody, jnp.zeros((eb, _HID), jnp.float32))
    msg = g * w
    agg = jnp.sum(msg.reshape(rb, k, _HID), axis=1)
    x3 = _ssp(jax.lax.dot_general(agg, l2w_ref[0], (((1,), (0,)), ((), ())),
                                  preferred_element_type=jnp.float32,
                                  precision=hp) + l2b_ref[0])
    hprev = jnp.where(l == 0, h0_ref[...], h_s[pl.ds(b * rb, rb), :])
    hn = hprev + jax.lax.dot_general(x3, lw_ref[0], (((1,), (0,)), ((), ())),
                                     preferred_element_type=jnp.float32,
                                     precision=hp) + lb_ref[0]
    hn_ref[...] = hn
    h_s[pl.ds(b * rb, rb), :] = hn
    x1n = jax.lax.dot_general(hn, l1n_ref[0], (((1,), (0,)), ((), ())),
                              preferred_element_type=jnp.float32,
                              precision=hp)
    x1h_s[wsl, pl.ds(b * rb, rb), :] = x1n.astype(jnp.bfloat16)


def _layers(scal, d_e, ti_e, ea_e, h, x1h, w1s, b1s, w2s, b2s, l2ws, l2bs,
            lws, lbs, l1s, np_, nblk, interpret=False):
    nl = _NL
    grid_spec = pltpu.PrefetchScalarGridSpec(
        num_scalar_prefetch=1,
        grid=(nl, nblk),
        in_specs=[
            pl.BlockSpec((_EB, 1), lambda l, b, s: (b, 0)),
            pl.BlockSpec((_EB, 1), lambda l, b, s: (b, 0)),
            pl.BlockSpec((_EB, _NGP), lambda l, b, s: (b, 0)),
            pl.BlockSpec((_RBL, _HID), lambda l, b, s: (b, 0)),
            pl.BlockSpec((np_, _HID), lambda l, b, s: (0, 0)),
            pl.BlockSpec((1, _NGP, _HID), lambda l, b, s: (l, 0, 0)),
            pl.BlockSpec((1, 1, _HID), lambda l, b, s: (l, 0, 0)),
            pl.BlockSpec((1, _HID, _HID), lambda l, b, s: (l, 0, 0)),
            pl.BlockSpec((1, 1, _HID), lambda l, b, s: (l, 0, 0)),
            pl.BlockSpec((1, _HID, _HID), lambda l, b, s: (l, 0, 0)),
            pl.BlockSpec((1, 1, _HID), lambda l, b, s: (l, 0, 0)),
            pl.BlockSpec((1, _HID, _HID), lambda l, b, s: (l, 0, 0)),
            pl.BlockSpec((1, 1, _HID), lambda l, b, s: (l, 0, 0)),
            pl.BlockSpec((1, _HID, _HID), lambda l, b, s: ((l + 1) % nl, 0, 0)),
        ],
        out_specs=[
            pl.BlockSpec((_RBL, _HID), lambda l, b, s: (b, 0)),
        ],
        scratch_shapes=[
            pltpu.VMEM((2, np_, _HID), jnp.bfloat16),
            pltpu.VMEM((np_, _HID), jnp.float32),
        ],
    )
    return pl.pallas_call(
        functools.partial(_layers_kernel, t=_TG, k=_K),
        grid_spec=grid_spec,
        out_shape=[
            jax.ShapeDtypeStruct((np_, _HID), jnp.float32),
        ],
        interpret=interpret,
    )(scal, d_e, ti_e, ea_e, h, x1h, w1s, b1s, w2s, b2s, l2ws, l2bs,
      lws, lbs, l1s)[0]


def _final_kernel(h_ref, pw_ref, pb_ref, g_ref, bb_ref, o_ref):
    y = jax.lax.dot_general(h_ref[...], pw_ref[...], (((1,), (0,)), ((), ())),
                            preferred_element_type=jnp.float32,
                            precision=jax.lax.Precision.HIGHEST) + pb_ref[...]
    mu = jnp.mean(y, axis=-1, keepdims=True)
    var = jnp.mean((y - mu) ** 2, axis=-1, keepdims=True)
    yn = (y - mu) / jnp.sqrt(var + 1e-5) * g_ref[...] + bb_ref[...]
    o_ref[...] = yn * jax.nn.sigmoid(yn)


def _final(h, pw, pb, g, bb, np_, nblk, interpret=False):
    return pl.pallas_call(
        _final_kernel,
        grid=(nblk,),
        in_specs=[
            pl.BlockSpec((_RB, _HID), lambda b: (b, 0)),
            pl.BlockSpec((_HID, _PROJ), lambda b: (0, 0)),
            pl.BlockSpec((1, _PROJ), lambda b: (0, 0)),
            pl.BlockSpec((1, _PROJ), lambda b: (0, 0)),
            pl.BlockSpec((1, _PROJ), lambda b: (0, 0)),
        ],
        out_specs=pl.BlockSpec((_RB, _PROJ), lambda b: (b, 0)),
        out_shape=jax.ShapeDtypeStruct((np_, _PROJ), jnp.float32),
        interpret=interpret,
    )(h, pw, pb[None, :], g[None, :], bb[None, :])


def _forward(z, pos, batch, emb, mlp_w1, mlp_b1, mlp_w2, mlp_b2, lin1_w,
             lin2_w, lin2_b, lin_w, lin_b, proj_w, proj_b, ln_g, ln_b,
             interpret=False):
    n = pos.shape[0]
    ng = mlp_w1.shape[1]
    offset = jnp.linspace(0.0, _CUTOFF, ng)
    import numpy as _np
    _step = float(_np.linspace(_np.float32(0.0), _np.float32(_CUTOFF), ng,
                               dtype=_np.float32)[1])
    coeff = -0.5 / _step ** 2
    offs = jnp.zeros((1, _NGP), jnp.float32).at[0, :ng].set(offset)

    topi, dist, ea_e, np_ = _build_edges(pos, batch, offs, coeff, interpret)
    nblk = np_ // _RB
    scal_l, nblk_l = _block_ranges(batch, n, np_, _RBL, _TG)
    d_e = dist.reshape(np_ * _K, 1)
    ti_e = topi.reshape(np_ * _K, 1)

    w1s = jnp.zeros((_NL, _NGP, _HID), jnp.float32).at[:, :ng].set(mlp_w1)
    w1s = w1s.astype(jnp.bfloat16)
    w2s = mlp_w2.astype(jnp.bfloat16)
    h, x1h = _init_h(z, emb, lin1_w[0], np_, nblk, interpret)
    h = _layers(scal_l, d_e, ti_e, ea_e, h, x1h, w1s,
                mlp_b1[:, None, :], w2s, mlp_b2[:, None, :],
                lin2_w, lin2_b[:, None, :], lin_w, lin_b[:, None, :],
                lin1_w, np_, nblk_l, interpret)
    out = _final(h, proj_w, proj_b, ln_g, ln_b, np_, nblk, interpret)
    return out[:n], batch


def kernel(z, pos, batch, emb, mlp_w1, mlp_b1, mlp_w2, mlp_b2, lin1_w,
           lin2_w, lin2_b, lin_w, lin_b, proj_w, proj_b, ln_g, ln_b):
    return _forward(z, pos, batch, emb, mlp_w1, mlp_b1, mlp_w2, mlp_b2,
                    lin1_w, lin2_w, lin2_b, lin_w, lin_b, proj_w, proj_b,
                    ln_g, ln_b)
